# baseline (speedup 1.0000x reference)
import functools

import jax
import jax.numpy as jnp
from jax import lax
from jax.experimental import pallas as pl
from jax.experimental.pallas import tpu as pltpu
from jax.experimental.pallas import tpu_sc as plsc

EPS = 1e-5
BLOCK_V = 2048


def _tc_pack(ltT, vocab, dim):
    pack = 128 // dim
    quarter = BLOCK_V // pack
    grid = pl.cdiv(vocab, BLOCK_V)

    def body(x_ref, out_ref):
        x = x_ref[...]
        ii = lax.broadcasted_iota(jnp.int32, (dim, dim), 0)
        jj = lax.broadcasted_iota(jnp.int32, (dim, dim), 1)
        eye = (ii == jj).astype(jnp.float32)
        for a in range(pack):
            xa = x[:, a * quarter:(a + 1) * quarter]
            out_ref[:, a * dim:(a + 1) * dim] = lax.dot_general(
                xa, eye, (((0,), (0,)), ((), ())),
                precision=lax.Precision.HIGHEST)

    return pl.pallas_call(
        body,
        grid=(grid,),
        in_specs=[pl.BlockSpec((dim, BLOCK_V), lambda i: (0, i))],
        out_specs=pl.BlockSpec((quarter, 128), lambda i: (i, 0)),
        out_shape=jax.ShapeDtypeStruct((grid * quarter, 128), jnp.float32),
    )(ltT)


def _sc_gather(num_rows, dim, tbl_rows, nbuf_chunks):
    info = plsc.get_sparse_core_info()
    nc, ns = info.num_cores, info.num_subcores
    nw = nc * ns
    rows_per_w = num_rows // nw
    assert num_rows % nw == 0 and rows_per_w % nbuf_chunks == 0
    chunk = rows_per_w // nbuf_chunks
    assert chunk % 16 == 0
    mesh = plsc.VectorSubcoreMesh(core_axis_name="c", subcore_axis_name="s")

    @functools.partial(
        pl.kernel,
        mesh=mesh,
        compiler_params=pltpu.CompilerParams(use_tc_tiling_on_sc=False),
        out_type=jax.ShapeDtypeStruct((num_rows, dim), jnp.float32),
        scratch_types=[
            pltpu.VMEM((2, chunk), jnp.int32),
            pltpu.VMEM((2, chunk), jnp.int32),
            pltpu.VMEM((2, chunk, dim), jnp.float32),
            pltpu.SemaphoreType.DMA,
            pltpu.SemaphoreType.DMA,
        ],
    )
    def gather(idx_hbm, tbl_hbm, out_hbm, idx_v, row_v, rows_v, gsem, osem):
        wid = lax.axis_index("s") * nc + lax.axis_index("c")
        base = wid * rows_per_w
        quarter = BLOCK_V // (128 // dim)
        copies = [None] * nbuf_chunks
        for k in range(nbuf_chunks):
            slot = k % 2
            pltpu.sync_copy(idx_hbm.at[pl.ds(base + k * chunk, chunk)],
                            idx_v.at[slot])
            for t in range(chunk // 16):
                v = idx_v[slot, pl.ds(t * 16, 16)]
                blk = lax.shift_right_logical(v, 11)
                rem = lax.bitwise_and(v, BLOCK_V - 1)
                r = lax.bitwise_and(rem, quarter - 1)
                a = lax.shift_right_logical(rem, 9)
                s = (blk * quarter + r) * 4 + a
                row_v[slot, pl.ds(t * 16, 16)] = s
            copies[k] = pltpu.async_copy(tbl_hbm.at[row_v.at[slot]],
                                         rows_v.at[slot], gsem)
            if k > 0:
                pltpu.async_copy(
                    rows_v.at[(k - 1) % 2],
                    out_hbm.at[pl.ds(base + (k - 1) * chunk, chunk)],
                    osem).wait()
            copies[k].wait()
        pltpu.async_copy(
            rows_v.at[(nbuf_chunks - 1) % 2],
            out_hbm.at[pl.ds(base + (nbuf_chunks - 1) * chunk, chunk)],
            osem).wait()

    return gather


def _tc_energy(e2, batch, npairs, dim, block_b):
    width = npairs * dim
    maxnorm = 1.0 - EPS

    def body(e_ref, out_ref):
        x = e_ref[...]
        kk = lax.broadcasted_iota(jnp.int32, (dim, width), 1)
        dd = lax.broadcasted_iota(jnp.int32, (dim, width), 0)
        bmat = (kk % dim == dd).astype(jnp.float32)
        km = lax.broadcasted_iota(jnp.int32, (width, npairs), 0)
        jm = lax.broadcasted_iota(jnp.int32, (width, npairs), 1)
        mmat = (km // dim == jm).astype(jnp.float32)

        x0 = x[:, :dim]
        y = lax.dot(x0, bmat, precision=lax.Precision.HIGHEST)
        ss = lax.dot(x * x, mmat, precision=lax.Precision.HIGHEST)
        dots = lax.dot(x * y, mmat, precision=lax.Precision.HIGHEST)

        norm = jnp.sqrt(ss)
        f = jnp.where(norm > maxnorm, maxnorm / jnp.maximum(norm, EPS), 1.0)
        ss_n = ss * f * f
        uu = ss_n[:, 0:1]
        vv = ss_n[:, 1:]
        f0 = f[:, 0:1]
        uv = uu + vv - 2.0 * f0 * f[:, 1:] * dots[:, 1:]
        alpha = jnp.clip(1.0 - uu, EPS, None)
        beta = jnp.clip(1.0 - vv, EPS, None)
        gamma = jnp.clip(1.0 + 2.0 * uv / (alpha * beta), 1.0 + EPS, None)
        out_ref[...] = jnp.log(gamma + jnp.sqrt((gamma - 1.0) * (gamma + 1.0)))

    grid = batch // block_b
    return pl.pallas_call(
        body,
        grid=(grid,),
        in_specs=[pl.BlockSpec((block_b, width), lambda i: (i, 0))],
        out_specs=pl.BlockSpec((block_b, npairs - 1), lambda i: (i, 0)),
        out_shape=jax.ShapeDtypeStruct((batch, npairs - 1), jnp.float32),
    )(e2)


def kernel(inputs, lt):
    batch, npairs = inputs.shape
    vocab, dim = lt.shape
    idx = inputs.reshape(batch * npairs)
    packed = _tc_pack(lt.T, vocab, dim)
    prow, _ = packed.shape
    tbl = packed.reshape(prow * 128).reshape(prow * 128 // dim, dim)
    e = _sc_gather(batch * npairs, dim, tbl.shape[0], nbuf_chunks=4)(idx, tbl)
    e2 = e.reshape(batch, npairs * dim)
    return _tc_energy(e2, batch, npairs, dim, block_b=512)


# speedup vs baseline: 1.3650x; 1.3650x over previous
import functools

import jax
import jax.numpy as jnp
from jax import lax
from jax.experimental import pallas as pl
from jax.experimental.pallas import tpu as pltpu
from jax.experimental.pallas import tpu_sc as plsc

EPS = 1e-5
BLOCK_V = 2048


def _tc_pack(ltT, vocab, dim):
    pack = 128 // dim
    quarter = BLOCK_V // pack
    grid = pl.cdiv(vocab, BLOCK_V)

    def body(x_ref, out_ref):
        x = x_ref[...]
        xt = x.T
        for a in range(pack):
            out_ref[:, a * dim:(a + 1) * dim] = (
                xt[a * quarter:(a + 1) * quarter, :])

    return pl.pallas_call(
        body,
        grid=(grid,),
        in_specs=[pl.BlockSpec((dim, BLOCK_V), lambda i: (0, i))],
        out_specs=pl.BlockSpec((quarter, 128), lambda i: (i, 0)),
        out_shape=jax.ShapeDtypeStruct((grid * quarter, 128), jnp.float32),
    )(ltT)


def _sc_gather(num_rows, dim, tbl_rows, nbuf_chunks):
    info = plsc.get_sparse_core_info()
    nc, ns = info.num_cores, info.num_subcores
    nw = nc * ns
    rows_per_w = num_rows // nw
    assert num_rows % nw == 0 and rows_per_w % nbuf_chunks == 0
    chunk = rows_per_w // nbuf_chunks
    assert chunk % 16 == 0
    mesh = plsc.VectorSubcoreMesh(core_axis_name="c", subcore_axis_name="s")

    @functools.partial(
        pl.kernel,
        mesh=mesh,
        compiler_params=pltpu.CompilerParams(use_tc_tiling_on_sc=False),
        out_type=jax.ShapeDtypeStruct((num_rows, dim), jnp.float32),
        scratch_types=[
            pltpu.VMEM((2, chunk), jnp.int32),
            pltpu.VMEM((2, chunk), jnp.int32),
            pltpu.VMEM((2, chunk, dim), jnp.float32),
            pltpu.SemaphoreType.DMA,
            pltpu.SemaphoreType.DMA,
        ],
    )
    def gather(idx_hbm, tbl_hbm, out_hbm, idx_v, row_v, rows_v, gsem, osem):
        wid = lax.axis_index("s") * nc + lax.axis_index("c")
        base = wid * rows_per_w
        quarter = BLOCK_V // (128 // dim)
        copies = [None] * nbuf_chunks
        for k in range(nbuf_chunks):
            slot = k % 2
            pltpu.sync_copy(idx_hbm.at[pl.ds(base + k * chunk, chunk)],
                            idx_v.at[slot])
            for t in range(chunk // 16):
                v = idx_v[slot, pl.ds(t * 16, 16)]
                blk = lax.shift_right_logical(v, 11)
                rem = lax.bitwise_and(v, BLOCK_V - 1)
                r = lax.bitwise_and(rem, quarter - 1)
                a = lax.shift_right_logical(rem, 9)
                s = (blk * quarter + r) * 4 + a
                row_v[slot, pl.ds(t * 16, 16)] = s
            copies[k] = pltpu.async_copy(tbl_hbm.at[row_v.at[slot]],
                                         rows_v.at[slot], gsem)
            if k > 0:
                pltpu.async_copy(
                    rows_v.at[(k - 1) % 2],
                    out_hbm.at[pl.ds(base + (k - 1) * chunk, chunk)],
                    osem).wait()
            copies[k].wait()
        pltpu.async_copy(
            rows_v.at[(nbuf_chunks - 1) % 2],
            out_hbm.at[pl.ds(base + (nbuf_chunks - 1) * chunk, chunk)],
            osem).wait()

    return gather


def _tc_energy(e2, bmat, mmat, batch, npairs, dim, block_b):
    width = npairs * dim
    maxnorm = 1.0 - EPS

    def body(e_ref, b_ref, m_ref, out_ref):
        x = e_ref[...]
        bmat_v = b_ref[...]
        mmat_v = m_ref[...]
        x0 = x[:, :dim]
        y = lax.dot(x0, bmat_v, precision=lax.Precision.HIGHEST)
        ss = lax.dot(x * x, mmat_v, precision=lax.Precision.HIGHEST)
        dots = lax.dot(x * y, mmat_v, precision=lax.Precision.HIGHEST)

        norm = jnp.sqrt(ss)
        f = jnp.where(norm > maxnorm, maxnorm / jnp.maximum(norm, EPS), 1.0)
        ss_n = ss * f * f
        uu = ss_n[:, 0:1]
        vv = ss_n[:, 1:]
        f0 = f[:, 0:1]
        uv = uu + vv - 2.0 * f0 * f[:, 1:] * dots[:, 1:]
        alpha = jnp.clip(1.0 - uu, EPS, None)
        beta = jnp.clip(1.0 - vv, EPS, None)
        gamma = jnp.clip(1.0 + 2.0 * uv / (alpha * beta), 1.0 + EPS, None)
        out_ref[...] = jnp.log(gamma + jnp.sqrt((gamma - 1.0) * (gamma + 1.0)))

    grid = batch // block_b
    return pl.pallas_call(
        body,
        grid=(grid,),
        in_specs=[
            pl.BlockSpec((block_b, width), lambda i: (i, 0)),
            pl.BlockSpec((dim, width), lambda i: (0, 0)),
            pl.BlockSpec((width, npairs), lambda i: (0, 0)),
        ],
        out_specs=pl.BlockSpec((block_b, npairs - 1), lambda i: (i, 0)),
        out_shape=jax.ShapeDtypeStruct((batch, npairs - 1), jnp.float32),
    )(e2, bmat, mmat)


def kernel(inputs, lt):
    batch, npairs = inputs.shape
    vocab, dim = lt.shape
    width = npairs * dim
    idx = inputs.reshape(batch * npairs)
    packed = _tc_pack(lt.T, vocab, dim)
    prow, _ = packed.shape
    tbl = packed.reshape(prow * 128).reshape(prow * 128 // dim, dim)
    e = _sc_gather(batch * npairs, dim, tbl.shape[0], nbuf_chunks=4)(idx, tbl)
    e2 = e.reshape(batch, width)
    kk = lax.broadcasted_iota(jnp.int32, (dim, width), 1)
    dd = lax.broadcasted_iota(jnp.int32, (dim, width), 0)
    bmat = (kk % dim == dd).astype(jnp.float32)
    km = lax.broadcasted_iota(jnp.int32, (width, npairs), 0)
    jm = lax.broadcasted_iota(jnp.int32, (width, npairs), 1)
    mmat = (km // dim == jm).astype(jnp.float32)
    return _tc_energy(e2, bmat, mmat, batch, npairs, dim, block_b=1024)


# speedup vs baseline: 1.7316x; 1.2685x over previous
import functools

import jax
import jax.numpy as jnp
from jax import lax
from jax.experimental import pallas as pl
from jax.experimental.pallas import tpu as pltpu
from jax.experimental.pallas import tpu_sc as plsc

EPS = 1e-5
BLOCK_V = 4096
_LOG_BLOCK_V = BLOCK_V.bit_length() - 1


def _tc_pack(ltT, vocab, dim):
    pack = 128 // dim
    quarter = BLOCK_V // pack
    grid = pl.cdiv(vocab, BLOCK_V)

    def body(x_ref, out_ref):
        x = x_ref[...]
        xt = x.T
        for a in range(pack):
            out_ref[:, a * dim:(a + 1) * dim] = (
                xt[a * quarter:(a + 1) * quarter, :])

    return pl.pallas_call(
        body,
        grid=(grid,),
        in_specs=[pl.BlockSpec((dim, BLOCK_V), lambda i: (0, i))],
        out_specs=pl.BlockSpec((quarter, 128), lambda i: (i, 0)),
        out_shape=jax.ShapeDtypeStruct((grid * quarter, 128), jnp.float32),
    )(ltT)


def _sc_gather(num_rows, dim, tbl_rows, nbuf_chunks):
    info = plsc.get_sparse_core_info()
    nc, ns = info.num_cores, info.num_subcores
    nw = nc * ns
    rows_per_w = num_rows // nw
    assert num_rows % nw == 0 and rows_per_w % nbuf_chunks == 0
    chunk = rows_per_w // nbuf_chunks
    assert chunk % 16 == 0
    mesh = plsc.VectorSubcoreMesh(core_axis_name="c", subcore_axis_name="s")

    @functools.partial(
        pl.kernel,
        mesh=mesh,
        compiler_params=pltpu.CompilerParams(use_tc_tiling_on_sc=False),
        out_type=jax.ShapeDtypeStruct((num_rows, dim), jnp.float32),
        scratch_types=[
            pltpu.VMEM((2, chunk), jnp.int32),
            pltpu.VMEM((2, chunk), jnp.int32),
            pltpu.VMEM((2, chunk, dim), jnp.float32),
            pltpu.SemaphoreType.DMA,
            pltpu.SemaphoreType.DMA,
        ],
    )
    def gather(idx_hbm, tbl_hbm, out_hbm, idx_v, row_v, rows_v, gsem, osem):
        wid = lax.axis_index("s") * nc + lax.axis_index("c")
        base = wid * rows_per_w
        quarter = BLOCK_V // (128 // dim)
        copies = [None] * nbuf_chunks
        for k in range(nbuf_chunks):
            slot = k % 2
            pltpu.sync_copy(idx_hbm.at[pl.ds(base + k * chunk, chunk)],
                            idx_v.at[slot])
            for t in range(chunk // 16):
                v = idx_v[slot, pl.ds(t * 16, 16)]
                blk = lax.shift_right_logical(v, _LOG_BLOCK_V)
                rem = lax.bitwise_and(v, BLOCK_V - 1)
                r = lax.bitwise_and(rem, quarter - 1)
                a = lax.shift_right_logical(rem, _LOG_BLOCK_V - 2)
                s = (blk * quarter + r) * 4 + a
                row_v[slot, pl.ds(t * 16, 16)] = s
            copies[k] = pltpu.async_copy(tbl_hbm.at[row_v.at[slot]],
                                         rows_v.at[slot], gsem)
            if k > 0:
                pltpu.async_copy(
                    rows_v.at[(k - 1) % 2],
                    out_hbm.at[pl.ds(base + (k - 1) * chunk, chunk)],
                    osem).wait()
            copies[k].wait()
        pltpu.async_copy(
            rows_v.at[(nbuf_chunks - 1) % 2],
            out_hbm.at[pl.ds(base + (nbuf_chunks - 1) * chunk, chunk)],
            osem).wait()

    return gather


def _tc_energy(e2, bmat, mmat, batch, npairs, dim, block_b):
    width = npairs * dim
    maxnorm = 1.0 - EPS

    def body(e_ref, b_ref, m_ref, out_ref):
        x = e_ref[...]
        bmat_v = b_ref[...]
        mmat_v = m_ref[...]
        x0 = x[:, :dim]
        y = lax.dot(x0, bmat_v, precision=lax.Precision.HIGHEST)
        ss = lax.dot(x * x, mmat_v, precision=lax.Precision.HIGHEST)
        dots = lax.dot(x * y, mmat_v, precision=lax.Precision.HIGHEST)

        norm = jnp.sqrt(ss)
        f = jnp.where(norm > maxnorm, maxnorm / jnp.maximum(norm, EPS), 1.0)
        ss_n = ss * f * f
        uu = ss_n[:, 0:1]
        vv = ss_n[:, 1:]
        f0 = f[:, 0:1]
        uv = uu + vv - 2.0 * f0 * f[:, 1:] * dots[:, 1:]
        alpha = jnp.clip(1.0 - uu, EPS, None)
        beta = jnp.clip(1.0 - vv, EPS, None)
        gamma = jnp.clip(1.0 + 2.0 * uv / (alpha * beta), 1.0 + EPS, None)
        out_ref[...] = jnp.log(gamma + jnp.sqrt((gamma - 1.0) * (gamma + 1.0)))

    grid = batch // block_b
    return pl.pallas_call(
        body,
        grid=(grid,),
        in_specs=[
            pl.BlockSpec((block_b, width), lambda i: (i, 0)),
            pl.BlockSpec((dim, width), lambda i: (0, 0)),
            pl.BlockSpec((width, npairs), lambda i: (0, 0)),
        ],
        out_specs=pl.BlockSpec((block_b, npairs - 1), lambda i: (i, 0)),
        out_shape=jax.ShapeDtypeStruct((batch, npairs - 1), jnp.float32),
    )(e2, bmat, mmat)


def kernel(inputs, lt):
    batch, npairs = inputs.shape
    vocab, dim = lt.shape
    width = npairs * dim
    idx = inputs.reshape(batch * npairs)
    packed = _tc_pack(lt.T, vocab, dim)
    prow, _ = packed.shape
    tbl = packed.reshape(prow * 128).reshape(prow * 128 // dim, dim)
    e = _sc_gather(batch * npairs, dim, tbl.shape[0], nbuf_chunks=4)(idx, tbl)
    e2 = e.reshape(batch, width)
    kk = lax.broadcasted_iota(jnp.int32, (dim, width), 1)
    dd = lax.broadcasted_iota(jnp.int32, (dim, width), 0)
    bmat = (kk % dim == dd).astype(jnp.float32)
    km = lax.broadcasted_iota(jnp.int32, (width, npairs), 0)
    jm = lax.broadcasted_iota(jnp.int32, (width, npairs), 1)
    mmat = (km // dim == jm).astype(jnp.float32)
    return _tc_energy(e2, bmat, mmat, batch, npairs, dim, block_b=1024)


# speedup vs baseline: 1.9187x; 1.1081x over previous
import functools

import jax
import jax.numpy as jnp
from jax import lax
from jax.experimental import pallas as pl
from jax.experimental.pallas import tpu as pltpu
from jax.experimental.pallas import tpu_sc as plsc

EPS = 1e-5
BLOCK_V = 8192
_LOG_BLOCK_V = BLOCK_V.bit_length() - 1


def _tc_pack(ltT, vocab, dim):
    pack = 128 // dim
    quarter = BLOCK_V // pack
    grid = pl.cdiv(vocab, BLOCK_V)

    def body(x_ref, out_ref):
        x = x_ref[...]
        xt = x.T
        for a in range(pack):
            out_ref[:, a * dim:(a + 1) * dim] = (
                xt[a * quarter:(a + 1) * quarter, :])

    return pl.pallas_call(
        body,
        grid=(grid,),
        in_specs=[pl.BlockSpec((dim, BLOCK_V), lambda i: (0, i))],
        out_specs=pl.BlockSpec((quarter, 128), lambda i: (i, 0)),
        out_shape=jax.ShapeDtypeStruct((grid * quarter, 128), jnp.float32),
    )(ltT)


def _sc_gather(num_rows, dim, tbl_rows, nbuf_chunks):
    info = plsc.get_sparse_core_info()
    nc, ns = info.num_cores, info.num_subcores
    nw = nc * ns
    rows_per_w = num_rows // nw
    assert num_rows % nw == 0 and rows_per_w % nbuf_chunks == 0
    chunk = rows_per_w // nbuf_chunks
    assert chunk % 16 == 0
    mesh = plsc.VectorSubcoreMesh(core_axis_name="c", subcore_axis_name="s")

    @functools.partial(
        pl.kernel,
        mesh=mesh,
        compiler_params=pltpu.CompilerParams(use_tc_tiling_on_sc=False),
        out_type=jax.ShapeDtypeStruct((num_rows, dim), jnp.float32),
        scratch_types=[
            pltpu.VMEM((2, chunk), jnp.int32),
            pltpu.VMEM((2, chunk), jnp.int32),
            pltpu.VMEM((2, chunk, dim), jnp.float32),
            pltpu.SemaphoreType.DMA,
            pltpu.SemaphoreType.DMA,
        ],
    )
    def gather(idx_hbm, tbl_hbm, out_hbm, idx_v, row_v, rows_v, gsem, osem):
        wid = lax.axis_index("s") * nc + lax.axis_index("c")
        base = wid * rows_per_w
        quarter = BLOCK_V // (128 // dim)
        copies = [None] * nbuf_chunks
        for k in range(nbuf_chunks):
            slot = k % 2
            pltpu.sync_copy(idx_hbm.at[pl.ds(base + k * chunk, chunk)],
                            idx_v.at[slot])
            for t in range(chunk // 16):
                v = idx_v[slot, pl.ds(t * 16, 16)]
                blk = lax.shift_right_logical(v, _LOG_BLOCK_V)
                rem = lax.bitwise_and(v, BLOCK_V - 1)
                r = lax.bitwise_and(rem, quarter - 1)
                a = lax.shift_right_logical(rem, _LOG_BLOCK_V - 2)
                s = (blk * quarter + r) * 4 + a
                row_v[slot, pl.ds(t * 16, 16)] = s
            copies[k] = pltpu.async_copy(tbl_hbm.at[row_v.at[slot]],
                                         rows_v.at[slot], gsem)
            if k > 0:
                pltpu.async_copy(
                    rows_v.at[(k - 1) % 2],
                    out_hbm.at[pl.ds(base + (k - 1) * chunk, chunk)],
                    osem).wait()
            copies[k].wait()
        pltpu.async_copy(
            rows_v.at[(nbuf_chunks - 1) % 2],
            out_hbm.at[pl.ds(base + (nbuf_chunks - 1) * chunk, chunk)],
            osem).wait()

    return gather


def _tc_energy(e2, bmat, mmat, batch, npairs, dim, block_b):
    width = npairs * dim
    maxnorm = 1.0 - EPS

    def body(e_ref, b_ref, m_ref, out_ref):
        x = e_ref[...]
        bmat_v = b_ref[...]
        mmat_v = m_ref[...]
        x0 = x[:, :dim]
        y = lax.dot(x0, bmat_v, precision=lax.Precision.HIGHEST)
        ss = lax.dot(x * x, mmat_v, precision=lax.Precision.HIGHEST)
        dots = lax.dot(x * y, mmat_v, precision=lax.Precision.HIGHEST)

        norm = jnp.sqrt(ss)
        f = jnp.where(norm > maxnorm, maxnorm / jnp.maximum(norm, EPS), 1.0)
        ss_n = ss * f * f
        uu = ss_n[:, 0:1]
        vv = ss_n[:, 1:]
        f0 = f[:, 0:1]
        uv = uu + vv - 2.0 * f0 * f[:, 1:] * dots[:, 1:]
        alpha = jnp.clip(1.0 - uu, EPS, None)
        beta = jnp.clip(1.0 - vv, EPS, None)
        gamma = jnp.clip(1.0 + 2.0 * uv / (alpha * beta), 1.0 + EPS, None)
        out_ref[...] = jnp.log(gamma + jnp.sqrt((gamma - 1.0) * (gamma + 1.0)))

    grid = batch // block_b
    return pl.pallas_call(
        body,
        grid=(grid,),
        in_specs=[
            pl.BlockSpec((block_b, width), lambda i: (i, 0)),
            pl.BlockSpec((dim, width), lambda i: (0, 0)),
            pl.BlockSpec((width, npairs), lambda i: (0, 0)),
        ],
        out_specs=pl.BlockSpec((block_b, npairs - 1), lambda i: (i, 0)),
        out_shape=jax.ShapeDtypeStruct((batch, npairs - 1), jnp.float32),
    )(e2, bmat, mmat)


def kernel(inputs, lt):
    batch, npairs = inputs.shape
    vocab, dim = lt.shape
    width = npairs * dim
    idx = inputs.reshape(batch * npairs)
    packed = _tc_pack(lt.T, vocab, dim)
    prow, _ = packed.shape
    tbl = packed.reshape(prow * 128).reshape(prow * 128 // dim, dim)
    e = _sc_gather(batch * npairs, dim, tbl.shape[0], nbuf_chunks=4)(idx, tbl)
    e2 = e.reshape(batch, width)
    kk = lax.broadcasted_iota(jnp.int32, (dim, width), 1)
    dd = lax.broadcasted_iota(jnp.int32, (dim, width), 0)
    bmat = (kk % dim == dd).astype(jnp.float32)
    km = lax.broadcasted_iota(jnp.int32, (width, npairs), 0)
    jm = lax.broadcasted_iota(jnp.int32, (width, npairs), 1)
    mmat = (km // dim == jm).astype(jnp.float32)
    return _tc_energy(e2, bmat, mmat, batch, npairs, dim, block_b=1024)


# speedup vs baseline: 1.9532x; 1.0180x over previous
import functools

import jax
import jax.numpy as jnp
from jax import lax
from jax.experimental import pallas as pl
from jax.experimental.pallas import tpu as pltpu
from jax.experimental.pallas import tpu_sc as plsc

EPS = 1e-5
BLOCK_V = 16384
_LOG_BLOCK_V = BLOCK_V.bit_length() - 1


def _tc_pack(ltT, vocab, dim):
    pack = 128 // dim
    quarter = BLOCK_V // pack
    grid = pl.cdiv(vocab, BLOCK_V)

    def body(x_ref, out_ref):
        x = x_ref[...]
        xt = x.T
        for a in range(pack):
            out_ref[:, a * dim:(a + 1) * dim] = (
                xt[a * quarter:(a + 1) * quarter, :])

    return pl.pallas_call(
        body,
        grid=(grid,),
        in_specs=[pl.BlockSpec((dim, BLOCK_V), lambda i: (0, i))],
        out_specs=pl.BlockSpec((quarter, 128), lambda i: (i, 0)),
        out_shape=jax.ShapeDtypeStruct((grid * quarter, 128), jnp.float32),
    )(ltT)


def _sc_gather(num_rows, dim, tbl_rows, nbuf_chunks):
    info = plsc.get_sparse_core_info()
    nc, ns = info.num_cores, info.num_subcores
    nw = nc * ns
    rows_per_w = num_rows // nw
    assert num_rows % nw == 0 and rows_per_w % nbuf_chunks == 0
    chunk = rows_per_w // nbuf_chunks
    assert chunk % 16 == 0
    mesh = plsc.VectorSubcoreMesh(core_axis_name="c", subcore_axis_name="s")

    @functools.partial(
        pl.kernel,
        mesh=mesh,
        compiler_params=pltpu.CompilerParams(use_tc_tiling_on_sc=False),
        out_type=jax.ShapeDtypeStruct((num_rows, dim), jnp.float32),
        scratch_types=[
            pltpu.VMEM((2, chunk), jnp.int32),
            pltpu.VMEM((2, chunk), jnp.int32),
            pltpu.VMEM((2, chunk, dim), jnp.float32),
            pltpu.SemaphoreType.DMA,
            pltpu.SemaphoreType.DMA,
        ],
    )
    def gather(idx_hbm, tbl_hbm, out_hbm, idx_v, row_v, rows_v, gsem, osem):
        wid = lax.axis_index("s") * nc + lax.axis_index("c")
        base = wid * rows_per_w
        quarter = BLOCK_V // (128 // dim)
        copies = [None] * nbuf_chunks
        for k in range(nbuf_chunks):
            slot = k % 2
            pltpu.sync_copy(idx_hbm.at[pl.ds(base + k * chunk, chunk)],
                            idx_v.at[slot])
            for t in range(chunk // 16):
                v = idx_v[slot, pl.ds(t * 16, 16)]
                blk = lax.shift_right_logical(v, _LOG_BLOCK_V)
                rem = lax.bitwise_and(v, BLOCK_V - 1)
                r = lax.bitwise_and(rem, quarter - 1)
                a = lax.shift_right_logical(rem, _LOG_BLOCK_V - 2)
                s = (blk * quarter + r) * 4 + a
                row_v[slot, pl.ds(t * 16, 16)] = s
            copies[k] = pltpu.async_copy(tbl_hbm.at[row_v.at[slot]],
                                         rows_v.at[slot], gsem)
            if k > 0:
                pltpu.async_copy(
                    rows_v.at[(k - 1) % 2],
                    out_hbm.at[pl.ds(base + (k - 1) * chunk, chunk)],
                    osem).wait()
            copies[k].wait()
        pltpu.async_copy(
            rows_v.at[(nbuf_chunks - 1) % 2],
            out_hbm.at[pl.ds(base + (nbuf_chunks - 1) * chunk, chunk)],
            osem).wait()

    return gather


def _tc_energy(e2, bmat, mmat, batch, npairs, dim, block_b):
    width = npairs * dim
    maxnorm = 1.0 - EPS

    def body(e_ref, b_ref, m_ref, out_ref):
        x = e_ref[...]
        bmat_v = b_ref[...]
        mmat_v = m_ref[...]
        x0 = x[:, :dim]
        y = lax.dot(x0, bmat_v, precision=lax.Precision.HIGHEST)
        ss = lax.dot(x * x, mmat_v, precision=lax.Precision.HIGHEST)
        dots = lax.dot(x * y, mmat_v, precision=lax.Precision.HIGHEST)

        norm = jnp.sqrt(ss)
        f = jnp.where(norm > maxnorm, maxnorm / jnp.maximum(norm, EPS), 1.0)
        ss_n = ss * f * f
        uu = ss_n[:, 0:1]
        vv = ss_n[:, 1:]
        f0 = f[:, 0:1]
        uv = uu + vv - 2.0 * f0 * f[:, 1:] * dots[:, 1:]
        alpha = jnp.clip(1.0 - uu, EPS, None)
        beta = jnp.clip(1.0 - vv, EPS, None)
        gamma = jnp.clip(1.0 + 2.0 * uv / (alpha * beta), 1.0 + EPS, None)
        out_ref[...] = jnp.log(gamma + jnp.sqrt((gamma - 1.0) * (gamma + 1.0)))

    grid = batch // block_b
    return pl.pallas_call(
        body,
        grid=(grid,),
        in_specs=[
            pl.BlockSpec((block_b, width), lambda i: (i, 0)),
            pl.BlockSpec((dim, width), lambda i: (0, 0)),
            pl.BlockSpec((width, npairs), lambda i: (0, 0)),
        ],
        out_specs=pl.BlockSpec((block_b, npairs - 1), lambda i: (i, 0)),
        out_shape=jax.ShapeDtypeStruct((batch, npairs - 1), jnp.float32),
    )(e2, bmat, mmat)


def kernel(inputs, lt):
    batch, npairs = inputs.shape
    vocab, dim = lt.shape
    width = npairs * dim
    idx = inputs.reshape(batch * npairs)
    packed = _tc_pack(lt.T, vocab, dim)
    prow, _ = packed.shape
    tbl = packed.reshape(prow * 128).reshape(prow * 128 // dim, dim)
    e = _sc_gather(batch * npairs, dim, tbl.shape[0], nbuf_chunks=4)(idx, tbl)
    e2 = e.reshape(batch, width)
    kk = lax.broadcasted_iota(jnp.int32, (dim, width), 1)
    dd = lax.broadcasted_iota(jnp.int32, (dim, width), 0)
    bmat = (kk % dim == dd).astype(jnp.float32)
    km = lax.broadcasted_iota(jnp.int32, (width, npairs), 0)
    jm = lax.broadcasted_iota(jnp.int32, (width, npairs), 1)
    mmat = (km // dim == jm).astype(jnp.float32)
    return _tc_energy(e2, bmat, mmat, batch, npairs, dim, block_b=1024)
